# baseline (speedup 1.0000x reference)
import functools

import jax
import jax.numpy as jnp
from jax import lax
from jax.experimental import pallas as pl
from jax.experimental.pallas import tpu as pltpu
from jax.experimental.pallas import tpu_sc as plsc

N_NODES = 10000
N_PAD = 10240
N_EDGES = 160000
NFEAT = 256
NHID = 512

NC = 2
NS = 16

_MESH = dict(core_axis_name="c", subcore_axis_name="s")
_SC_LINEAR = pltpu.CompilerParams(use_tc_tiling_on_sc=False)

_ROWS_PT = N_PAD // NS
_ZCH = 128


def _zero_acc(zeros_hbm, zbuf, acc, sid):
    pltpu.sync_copy(zeros_hbm, zbuf)
    for z in range(_ROWS_PT // _ZCH):
        pltpu.sync_copy(zbuf, acc.at[pl.ds(sid * _ROWS_PT + z * _ZCH, _ZCH)])


def _read_acc(out_hbm, zbuf, acc, cid, sid):
    for z in range(_ROWS_PT // _ZCH):
        r0 = sid * _ROWS_PT + z * _ZCH
        pltpu.sync_copy(acc.at[pl.ds(r0, _ZCH)], zbuf)
        pltpu.sync_copy(zbuf, out_hbm.at[cid].at[pl.ds(r0, _ZCH)])


def _edge_loop(n_chunks, gather_tab, src_all, dst_all, rows, gsem, acc):
    def start_gather(i, b):
        if gather_tab is not None:
            pltpu.async_copy(gather_tab.at[src_all.at[i]], rows[b], gsem[b])

    def wait_gather(i, b):
        if gather_tab is not None:
            pltpu.make_async_copy(gather_tab.at[src_all.at[i]], rows[b],
                                  gsem[b]).wait()

    def scatter(i, b):
        pltpu.sync_copy(rows[b], acc.at[dst_all.at[i]], add=True)

    start_gather(0, 0)

    def body(io, carry):
        for k in range(2):
            i = io * 2 + k
            start_gather(i + 1, (k + 1) % 2)
            wait_gather(i, k)
            scatter(i, k)
        return carry

    lax.fori_loop(0, (n_chunks - 1) // 2, body, 0)
    i_last = n_chunks - 1
    wait_gather(i_last, i_last % 2)
    scatter(i_last, i_last % 2)


_H_CH = 40
_H_EDGES_PT = N_EDGES // (NC * NS)
_H_CHUNKS = _H_EDGES_PT // _H_CH


def _deg_body(dst_hbm, ones_hbm, zeros_hbm, out_hbm,
              dst_all, ones_v, zbuf, acc, sem):
    cid = lax.axis_index("c")
    sid = lax.axis_index("s")
    _zero_acc(zeros_hbm, zbuf, acc, sid)
    pltpu.sync_copy(ones_hbm, ones_v)
    pltpu.sync_copy(dst_hbm.at[cid].at[sid], dst_all)
    plsc.subcore_barrier()

    def body(i, carry):
        pltpu.sync_copy(ones_v, acc.at[dst_all.at[i]], add=True)
        return carry

    lax.fori_loop(0, _H_CHUNKS, body, 0)
    plsc.subcore_barrier()
    _read_acc(out_hbm, zbuf, acc, cid, sid)


_deg_kernel = functools.partial(
    pl.kernel,
    out_type=jax.ShapeDtypeStruct((NC, N_PAD, 16), jnp.float32),
    mesh=plsc.VectorSubcoreMesh(**_MESH),
    compiler_params=_SC_LINEAR,
    scratch_types=[
        pltpu.VMEM((_H_CHUNKS, _H_CH), jnp.int32),
        pltpu.VMEM((_H_CH, 16), jnp.float32),
        pltpu.VMEM((_ZCH, 16), jnp.float32),
        pltpu.VMEM_SHARED((N_PAD, 16), jnp.float32),
        pltpu.SemaphoreType.DMA,
    ],
)(_deg_body)


def _sp_body(src_hbm, dst_hbm, norm16_hbm, zeros_hbm, out_hbm,
             src_all, dst_all, rows0, rows1, zbuf, acc, g0, g1):
    cid = lax.axis_index("c")
    sid = lax.axis_index("s")
    _zero_acc(zeros_hbm, zbuf, acc, sid)
    pltpu.sync_copy(src_hbm.at[cid].at[sid], src_all)
    pltpu.sync_copy(dst_hbm.at[cid].at[sid], dst_all)
    plsc.subcore_barrier()
    _edge_loop(_H_CHUNKS, norm16_hbm, src_all, dst_all,
               (rows0, rows1), (g0, g1), acc)
    plsc.subcore_barrier()
    _read_acc(out_hbm, zbuf, acc, cid, sid)


_sp_kernel = functools.partial(
    pl.kernel,
    out_type=jax.ShapeDtypeStruct((NC, N_PAD, 16), jnp.float32),
    mesh=plsc.VectorSubcoreMesh(**_MESH),
    compiler_params=_SC_LINEAR,
    scratch_types=[
        pltpu.VMEM((_H_CHUNKS, _H_CH), jnp.int32),
        pltpu.VMEM((_H_CHUNKS, _H_CH), jnp.int32),
        pltpu.VMEM((_H_CH, 16), jnp.float32),
        pltpu.VMEM((_H_CH, 16), jnp.float32),
        pltpu.VMEM((_ZCH, 16), jnp.float32),
        pltpu.VMEM_SHARED((N_PAD, 16), jnp.float32),
        pltpu.SemaphoreType.DMA,
        pltpu.SemaphoreType.DMA,
    ],
)(_sp_body)


_P_CH = 80
_P_EDGES_PT = N_EDGES // NS
_P_CHUNKS = _P_EDGES_PT // _P_CH
_W = 128


def _prop_body(table_hbm, srcx_hbm, dst_hbm, zeros_hbm, scale_hbm, out_hbm,
               dst_all, src0, src1, rows0, rows1, sbuf, acc, s0, s1, g0, g1):
    cid = lax.axis_index("c")
    sid = lax.axis_index("s")
    srcs = (src0, src1)
    rows = (rows0, rows1)
    ssem = (s0, s1)
    gsem = (g0, g1)

    pltpu.sync_copy(zeros_hbm, rows0)
    for z in range(_ROWS_PT // _P_CH):
        pltpu.sync_copy(rows0, acc.at[pl.ds(sid * _ROWS_PT + z * _P_CH, _P_CH)])
    pltpu.sync_copy(dst_hbm.at[sid], dst_all)
    plsc.subcore_barrier()

    def src_slice(i):
        base = cid * N_EDGES + sid * _P_EDGES_PT + i * _P_CH
        return srcx_hbm.at[pl.ds(base, _P_CH)]

    def start_src(i, b):
        pltpu.async_copy(src_slice(i), srcs[b], ssem[b])

    def wait_src(i, b):
        pltpu.make_async_copy(src_slice(i), srcs[b], ssem[b]).wait()

    def start_gather(b):
        pltpu.async_copy(table_hbm.at[srcs[b]], rows[b], gsem[b])

    def wait_gather(b):
        pltpu.make_async_copy(table_hbm.at[srcs[b]], rows[b], gsem[b]).wait()

    def scatter(i, b):
        pltpu.sync_copy(rows[b], acc.at[dst_all.at[i]], add=True)

    start_src(0, 0)
    wait_src(0, 0)
    start_gather(0)
    start_src(1, 1)

    def body(io, carry):
        for k in range(2):
            i = io * 2 + k
            b = k
            wait_src(i + 1, 1 - b)
            start_gather(1 - b)
            wait_gather(b)
            start_src(i + 2, b)
            scatter(i, b)
        return carry

    lax.fori_loop(0, (_P_CHUNKS - 3) // 2, body, 0)
    i = _P_CHUNKS - 3
    wait_src(i + 1, 1)
    start_gather(1)
    wait_gather(0)
    start_src(i + 2, 0)
    scatter(i, 0)
    wait_src(i + 2, 0)
    start_gather(0)
    wait_gather(1)
    scatter(i + 1, 1)
    wait_gather(0)
    scatter(i + 2, 0)

    plsc.subcore_barrier()
    for z in range(_ROWS_PT // _P_CH):
        r0 = sid * _ROWS_PT + z * _P_CH
        pltpu.sync_copy(acc.at[pl.ds(r0, _P_CH)], rows0)
        pltpu.sync_copy(scale_hbm.at[pl.ds(r0, _P_CH)], sbuf)

        def scale_rows(ro, carry):
            r0g = ro * 8
            for k in range(8):
                r = r0g + k
                sv = sbuf[r, :]
                for j in range(_W // 16):
                    cs = pl.ds(j * 16, 16)
                    rows0[r, cs] = rows0[r, cs] * sv
            return carry

        lax.fori_loop(0, _P_CH // 8, scale_rows, 0)
        pltpu.sync_copy(rows0, out_hbm.at[cid].at[pl.ds(r0, _P_CH)])


_prop_kernel = functools.partial(
    pl.kernel,
    out_type=jax.ShapeDtypeStruct((NC, N_PAD, _W), jnp.float32),
    mesh=plsc.VectorSubcoreMesh(**_MESH),
    scratch_types=[
        pltpu.VMEM((_P_CHUNKS, _P_CH), jnp.int32),
        pltpu.VMEM((_P_CH,), jnp.int32),
        pltpu.VMEM((_P_CH,), jnp.int32),
        pltpu.VMEM((_P_CH, _W), jnp.float32),
        pltpu.VMEM((_P_CH, _W), jnp.float32),
        pltpu.VMEM((_P_CH, 16), jnp.float32),
        pltpu.VMEM_SHARED((N_PAD, _W), jnp.float32),
        pltpu.SemaphoreType.DMA,
        pltpu.SemaphoreType.DMA,
        pltpu.SemaphoreType.DMA,
        pltpu.SemaphoreType.DMA,
    ],
)(_prop_body)


_RB = 2048
_RBF = 2000


def _norm_from_deg(deg_ref):
    deg = deg_ref[0, :, 0:1] + deg_ref[1, :, 0:1]
    return lax.rsqrt(jnp.maximum(deg, 1.0))


def _pre_body(x_ref, deg_ref, t_ref, n_ref, n2_ref):
    norm = _norm_from_deg(deg_ref)
    y = x_ref[...] * norm
    t_ref[0] = y[:, 0:128]
    t_ref[1] = y[:, 128:256]
    n_ref[...] = jnp.broadcast_to(norm, (norm.shape[0], 16))
    n2_ref[...] = jnp.broadcast_to(norm * norm, (norm.shape[0], 16))


def _pre_call(x, deg2):
    return pl.pallas_call(
        _pre_body,
        grid=(N_PAD // _RB,),
        in_specs=[
            pl.BlockSpec((_RB, NFEAT), lambda i: (i, 0)),
            pl.BlockSpec((NC, _RB, 16), lambda i: (0, i, 0)),
        ],
        out_specs=[
            pl.BlockSpec((NC, _RB, 128), lambda i: (0, i, 0)),
            pl.BlockSpec((_RB, 16), lambda i: (i, 0)),
            pl.BlockSpec((_RB, 16), lambda i: (i, 0)),
        ],
        out_shape=[
            jax.ShapeDtypeStruct((NC, N_PAD, 128), jnp.float32),
            jax.ShapeDtypeStruct((N_PAD, 16), jnp.float32),
            jax.ShapeDtypeStruct((N_PAD, 16), jnp.float32),
        ],
    )(x, deg2)


def _wcomb_body(a_ref, w2_ref, o_ref):
    o_ref[...] = jnp.dot(a_ref[...], w2_ref[...],
                         preferred_element_type=jnp.float32)


def _wcomb_call(a_pad, w2):
    return pl.pallas_call(
        _wcomb_body,
        out_shape=jax.ShapeDtypeStruct((264, NHID), jnp.float32),
    )(a_pad, w2)


def _fin_body(z2_ref, sp_ref, deg_ref, w_ref, b2_ref, o_ref):
    norm = _norm_from_deg(deg_ref)
    u = norm * (sp_ref[0, :, 0:1] + sp_ref[1, :, 0:1])
    acc = jnp.dot(z2_ref[0], w_ref[0:128], preferred_element_type=jnp.float32)
    acc = acc + jnp.dot(z2_ref[1], w_ref[128:256],
                        preferred_element_type=jnp.float32)
    acc = acc + u * w_ref[256:257]
    o_ref[...] = acc + b2_ref[...]


def _fin_call(z2, sp2, deg2, w_comb, b2):
    return pl.pallas_call(
        _fin_body,
        grid=(N_NODES // _RBF,),
        in_specs=[
            pl.BlockSpec((NC, _RBF, 128), lambda i: (0, i, 0)),
            pl.BlockSpec((NC, _RBF, 16), lambda i: (0, i, 0)),
            pl.BlockSpec((NC, _RBF, 16), lambda i: (0, i, 0)),
            pl.BlockSpec((264, NHID), lambda i: (0, 0)),
            pl.BlockSpec((1, NHID), lambda i: (0, 0)),
        ],
        out_specs=pl.BlockSpec((_RBF, NHID), lambda i: (i, 0)),
        out_shape=jax.ShapeDtypeStruct((N_NODES, NHID), jnp.float32),
    )(z2, sp2, deg2, w_comb, b2)


def kernel(x, edge_index, W1, b1, W2, b2):
    src = edge_index[0].astype(jnp.int32)
    dst = edge_index[1].astype(jnp.int32)

    ones16 = jnp.ones((_H_CH, 16), jnp.float32)
    zeros16 = jnp.zeros((_ZCH, 16), jnp.float32)
    zeros128 = jnp.zeros((_P_CH, _W), jnp.float32)

    srcx = jnp.concatenate([src, src + N_PAD])
    dstp = dst.reshape(NS, _P_CHUNKS, _P_CH)
    srch = src.reshape(NC, NS, _H_CHUNKS, _H_CH)
    dsth = dst.reshape(NC, NS, _H_CHUNKS, _H_CH)

    deg2 = _deg_kernel(dsth, ones16, zeros16)

    x_pad = jnp.pad(x, ((0, N_PAD - N_NODES), (0, 0)))
    table1, norm16, norm2_16 = _pre_call(x_pad, deg2)
    z1 = _prop_kernel(table1.reshape(NC * N_PAD, _W), srcx, dstp, zeros128,
                      norm2_16)
    sp2 = _sp_kernel(srch, dsth, norm16, zeros16)

    z2 = _prop_kernel(z1.reshape(NC * N_PAD, _W), srcx, dstp, zeros128,
                      norm16)

    a_pad = jnp.concatenate(
        [W1, b1[None, :], jnp.zeros((7, NHID), jnp.float32)], axis=0)
    w_comb = _wcomb_call(a_pad, W2)

    return _fin_call(z2, sp2, deg2, w_comb, b2[None, :])


# speedup vs baseline: 1.0246x; 1.0246x over previous
import functools

import jax
import jax.numpy as jnp
from jax import lax
from jax.experimental import pallas as pl
from jax.experimental.pallas import tpu as pltpu
from jax.experimental.pallas import tpu_sc as plsc

N_NODES = 10000
N_PAD = 10240
N_EDGES = 160000
NFEAT = 256
NHID = 512

NC = 2
NS = 16

_MESH = dict(core_axis_name="c", subcore_axis_name="s")
_SC_LINEAR = pltpu.CompilerParams(use_tc_tiling_on_sc=False)

_ROWS_PT = N_PAD // NS
_ZCH = 128


def _zero_acc(zeros_hbm, zbuf, acc, sid):
    pltpu.sync_copy(zeros_hbm, zbuf)
    for z in range(_ROWS_PT // _ZCH):
        pltpu.sync_copy(zbuf, acc.at[pl.ds(sid * _ROWS_PT + z * _ZCH, _ZCH)])


def _read_acc(out_hbm, zbuf, acc, cid, sid):
    for z in range(_ROWS_PT // _ZCH):
        r0 = sid * _ROWS_PT + z * _ZCH
        pltpu.sync_copy(acc.at[pl.ds(r0, _ZCH)], zbuf)
        pltpu.sync_copy(zbuf, out_hbm.at[cid].at[pl.ds(r0, _ZCH)])


def _edge_loop(n_chunks, gather_tab, src_all, dst_all, rows, gsem, ssem, acc):
    def start_gather(i, b):
        pltpu.async_copy(gather_tab.at[src_all.at[i]], rows[b], gsem[b])

    def wait_gather(i, b):
        pltpu.make_async_copy(gather_tab.at[src_all.at[i]], rows[b],
                              gsem[b]).wait()

    def start_scat(i, b):
        pltpu.async_copy(rows[b], acc.at[dst_all.at[i]], ssem[b], add=True)

    def wait_scat(i, b):
        pltpu.make_async_copy(rows[b], acc.at[dst_all.at[i]], ssem[b]).wait()

    start_gather(0, 0)
    start_gather(1, 1)
    wait_gather(0, 0)
    start_scat(0, 0)

    def body(io, carry):
        for k in range(2):
            i = io * 2 + 1 + k
            b = (1 + k) % 2
            wait_scat(i - 1, 1 - b)
            start_gather(i + 1, 1 - b)
            wait_gather(i, b)
            start_scat(i, b)
        return carry

    lax.fori_loop(0, (n_chunks - 3) // 2, body, 0)
    i = n_chunks - 2
    b = i % 2
    wait_scat(i - 1, 1 - b)
    start_gather(i + 1, 1 - b)
    wait_gather(i, b)
    start_scat(i, b)
    i = n_chunks - 1
    b = i % 2
    wait_gather(i, b)
    start_scat(i, b)
    wait_scat(i - 1, 1 - b)
    wait_scat(i, b)


_H_CH = 40
_H_EDGES_PT = N_EDGES // (NC * NS)
_H_CHUNKS = _H_EDGES_PT // _H_CH


def _deg_body(dst_hbm, ones_hbm, zeros_hbm, out_hbm,
              dst_all, ones_v, zbuf, acc, sem0, sem1):
    cid = lax.axis_index("c")
    sid = lax.axis_index("s")
    _zero_acc(zeros_hbm, zbuf, acc, sid)
    pltpu.sync_copy(ones_hbm, ones_v)
    pltpu.sync_copy(dst_hbm.at[cid].at[sid], dst_all)
    plsc.subcore_barrier()
    sems = (sem0, sem1)

    def start_sc(i, b):
        pltpu.async_copy(ones_v, acc.at[dst_all.at[i]], sems[b], add=True)

    def wait_sc(i, b):
        pltpu.make_async_copy(ones_v, acc.at[dst_all.at[i]], sems[b]).wait()

    start_sc(0, 0)
    start_sc(1, 1)

    def body(io, carry):
        for k in range(2):
            i = io * 2 + 2 + k
            b = k % 2
            wait_sc(i - 2, b)
            start_sc(i, b)
        return carry

    lax.fori_loop(0, (_H_CHUNKS - 3) // 2, body, 0)
    i = _H_CHUNKS - 1
    wait_sc(i - 2, i % 2)
    start_sc(i, i % 2)
    wait_sc(i - 1, (i - 1) % 2)
    wait_sc(i, i % 2)
    plsc.subcore_barrier()
    _read_acc(out_hbm, zbuf, acc, cid, sid)


_deg_kernel = functools.partial(
    pl.kernel,
    out_type=jax.ShapeDtypeStruct((NC, N_PAD, 16), jnp.float32),
    mesh=plsc.VectorSubcoreMesh(**_MESH),
    compiler_params=_SC_LINEAR,
    scratch_types=[
        pltpu.VMEM((_H_CHUNKS, _H_CH), jnp.int32),
        pltpu.VMEM((_H_CH, 16), jnp.float32),
        pltpu.VMEM((_ZCH, 16), jnp.float32),
        pltpu.VMEM_SHARED((N_PAD, 16), jnp.float32),
        pltpu.SemaphoreType.DMA,
        pltpu.SemaphoreType.DMA,
    ],
)(_deg_body)


def _sp_body(src_hbm, dst_hbm, norm16_hbm, zeros_hbm, out_hbm,
             src_all, dst_all, rows0, rows1, zbuf, acc, g0, g1, sc0, sc1):
    cid = lax.axis_index("c")
    sid = lax.axis_index("s")
    _zero_acc(zeros_hbm, zbuf, acc, sid)
    pltpu.sync_copy(src_hbm.at[cid].at[sid], src_all)
    pltpu.sync_copy(dst_hbm.at[cid].at[sid], dst_all)
    plsc.subcore_barrier()
    _edge_loop(_H_CHUNKS, norm16_hbm, src_all, dst_all,
               (rows0, rows1), (g0, g1), (sc0, sc1), acc)
    plsc.subcore_barrier()
    _read_acc(out_hbm, zbuf, acc, cid, sid)


_sp_kernel = functools.partial(
    pl.kernel,
    out_type=jax.ShapeDtypeStruct((NC, N_PAD, 16), jnp.float32),
    mesh=plsc.VectorSubcoreMesh(**_MESH),
    compiler_params=_SC_LINEAR,
    scratch_types=[
        pltpu.VMEM((_H_CHUNKS, _H_CH), jnp.int32),
        pltpu.VMEM((_H_CHUNKS, _H_CH), jnp.int32),
        pltpu.VMEM((_H_CH, 16), jnp.float32),
        pltpu.VMEM((_H_CH, 16), jnp.float32),
        pltpu.VMEM((_ZCH, 16), jnp.float32),
        pltpu.VMEM_SHARED((N_PAD, 16), jnp.float32),
        pltpu.SemaphoreType.DMA,
        pltpu.SemaphoreType.DMA,
        pltpu.SemaphoreType.DMA,
        pltpu.SemaphoreType.DMA,
    ],
)(_sp_body)


_P_CH = 80
_P_EDGES_PT = N_EDGES // NS
_P_CHUNKS = _P_EDGES_PT // _P_CH
_W = 128


def _prop_body(table_hbm, srcx_hbm, dst_hbm, zeros_hbm, scale_hbm, out_hbm,
               dst_all, src0, src1, rows0, rows1, sbuf, acc,
               s0, s1, g0, g1, c0, c1):
    cid = lax.axis_index("c")
    sid = lax.axis_index("s")
    srcs = (src0, src1)
    rows = (rows0, rows1)
    ssem = (s0, s1)
    gsem = (g0, g1)
    csem = (c0, c1)
    NZ = _ROWS_PT // _P_CH

    def zslice(z):
        return acc.at[pl.ds(sid * _ROWS_PT + z * _P_CH, _P_CH)]

    pltpu.sync_copy(zeros_hbm, rows0)
    for z in range(NZ):
        pltpu.async_copy(rows0, zslice(z), g0)
    pltpu.sync_copy(dst_hbm.at[sid], dst_all)
    for z in range(NZ):
        pltpu.make_async_copy(rows0, zslice(z), g0).wait()
    plsc.subcore_barrier()

    def src_slice(i):
        base = cid * N_EDGES + sid * _P_EDGES_PT + i * _P_CH
        return srcx_hbm.at[pl.ds(base, _P_CH)]

    def start_src(i, b):
        pltpu.async_copy(src_slice(i), srcs[b], ssem[b])

    def wait_src(i, b):
        pltpu.make_async_copy(src_slice(i), srcs[b], ssem[b]).wait()

    def start_gather(b):
        pltpu.async_copy(table_hbm.at[srcs[b]], rows[b], gsem[b])

    def wait_gather(b):
        pltpu.make_async_copy(table_hbm.at[srcs[b]], rows[b], gsem[b]).wait()

    def start_scat(i, b):
        pltpu.async_copy(rows[b], acc.at[dst_all.at[i]], csem[b], add=True)

    def wait_scat(i, b):
        pltpu.make_async_copy(rows[b], acc.at[dst_all.at[i]], csem[b]).wait()

    start_src(0, 0)
    wait_src(0, 0)
    start_gather(0)
    start_src(1, 1)
    wait_src(1, 1)
    start_gather(1)
    wait_gather(0)
    start_src(2, 0)
    start_scat(0, 0)

    def body(io, carry):
        for k in range(2):
            i = io * 2 + 1 + k
            b = (1 + k) % 2
            wait_src(i + 1, 1 - b)
            wait_scat(i - 1, 1 - b)
            start_gather(1 - b)
            wait_gather(b)
            start_src(i + 2, b)
            start_scat(i, b)
        return carry

    lax.fori_loop(0, (_P_CHUNKS - 3) // 2, body, 0)
    i = _P_CHUNKS - 2
    b = i % 2
    wait_src(i + 1, 1 - b)
    wait_scat(i - 1, 1 - b)
    start_gather(1 - b)
    wait_gather(b)
    start_scat(i, b)
    i = _P_CHUNKS - 1
    b = i % 2
    wait_gather(b)
    start_scat(i, b)
    wait_scat(i - 1, 1 - b)
    wait_scat(i, b)
    plsc.subcore_barrier()

    def oslice(z):
        return out_hbm.at[cid].at[pl.ds(sid * _ROWS_PT + z * _P_CH, _P_CH)]

    def scale_chunk(b):
        def scale_rows(ro, carry):
            for k in range(8):
                r = ro * 8 + k
                sv = sbuf[r, :]
                for j in range(_W // 16):
                    cs = pl.ds(j * 16, 16)
                    rows[b][r, cs] = rows[b][r, cs] * sv
            return carry
        lax.fori_loop(0, _P_CH // 8, scale_rows, 0)

    for z in range(NZ):
        b = z % 2
        if z >= 2:
            pltpu.make_async_copy(rows[b], oslice(z - 2), gsem[b]).wait()
        pltpu.sync_copy(zslice(z), rows[b])
        pltpu.sync_copy(scale_hbm.at[pl.ds(sid * _ROWS_PT + z * _P_CH, _P_CH)],
                        sbuf)
        scale_chunk(b)
        pltpu.async_copy(rows[b], oslice(z), gsem[b])
    pltpu.make_async_copy(rows[0], oslice(NZ - 2), gsem[0]).wait()
    pltpu.make_async_copy(rows[1], oslice(NZ - 1), gsem[1]).wait()


_prop_kernel = functools.partial(
    pl.kernel,
    out_type=jax.ShapeDtypeStruct((NC, N_PAD, _W), jnp.float32),
    mesh=plsc.VectorSubcoreMesh(**_MESH),
    scratch_types=[
        pltpu.VMEM((_P_CHUNKS, _P_CH), jnp.int32),
        pltpu.VMEM((_P_CH,), jnp.int32),
        pltpu.VMEM((_P_CH,), jnp.int32),
        pltpu.VMEM((_P_CH, _W), jnp.float32),
        pltpu.VMEM((_P_CH, _W), jnp.float32),
        pltpu.VMEM((_P_CH, 16), jnp.float32),
        pltpu.VMEM_SHARED((N_PAD, _W), jnp.float32),
        pltpu.SemaphoreType.DMA,
        pltpu.SemaphoreType.DMA,
        pltpu.SemaphoreType.DMA,
        pltpu.SemaphoreType.DMA,
        pltpu.SemaphoreType.DMA,
        pltpu.SemaphoreType.DMA,
    ],
)(_prop_body)


_RB = 2048
_RBF = 2000


def _norm_from_deg(deg_ref):
    deg = deg_ref[0, :, 0:1] + deg_ref[1, :, 0:1]
    return lax.rsqrt(jnp.maximum(deg, 1.0))


def _pre_body(x_ref, deg_ref, t_ref, n_ref, n2_ref):
    norm = _norm_from_deg(deg_ref)
    y = x_ref[...] * norm
    t_ref[0] = y[:, 0:128]
    t_ref[1] = y[:, 128:256]
    n_ref[...] = jnp.broadcast_to(norm, (norm.shape[0], 16))
    n2_ref[...] = jnp.broadcast_to(norm * norm, (norm.shape[0], 16))


def _pre_call(x, deg2):
    return pl.pallas_call(
        _pre_body,
        grid=(N_PAD // _RB,),
        in_specs=[
            pl.BlockSpec((_RB, NFEAT), lambda i: (i, 0)),
            pl.BlockSpec((NC, _RB, 16), lambda i: (0, i, 0)),
        ],
        out_specs=[
            pl.BlockSpec((NC, _RB, 128), lambda i: (0, i, 0)),
            pl.BlockSpec((_RB, 16), lambda i: (i, 0)),
            pl.BlockSpec((_RB, 16), lambda i: (i, 0)),
        ],
        out_shape=[
            jax.ShapeDtypeStruct((NC, N_PAD, 128), jnp.float32),
            jax.ShapeDtypeStruct((N_PAD, 16), jnp.float32),
            jax.ShapeDtypeStruct((N_PAD, 16), jnp.float32),
        ],
    )(x, deg2)


def _wcomb_body(a_ref, w2_ref, o_ref):
    o_ref[...] = jnp.dot(a_ref[...], w2_ref[...],
                         preferred_element_type=jnp.float32)


def _wcomb_call(a_pad, w2):
    return pl.pallas_call(
        _wcomb_body,
        out_shape=jax.ShapeDtypeStruct((264, NHID), jnp.float32),
    )(a_pad, w2)


def _fin_body(z2_ref, sp_ref, deg_ref, w_ref, b2_ref, o_ref):
    norm = _norm_from_deg(deg_ref)
    u = norm * (sp_ref[0, :, 0:1] + sp_ref[1, :, 0:1])
    acc = jnp.dot(z2_ref[0], w_ref[0:128], preferred_element_type=jnp.float32)
    acc = acc + jnp.dot(z2_ref[1], w_ref[128:256],
                        preferred_element_type=jnp.float32)
    acc = acc + u * w_ref[256:257]
    o_ref[...] = acc + b2_ref[...]


def _fin_call(z2, sp2, deg2, w_comb, b2):
    return pl.pallas_call(
        _fin_body,
        grid=(N_NODES // _RBF,),
        in_specs=[
            pl.BlockSpec((NC, _RBF, 128), lambda i: (0, i, 0)),
            pl.BlockSpec((NC, _RBF, 16), lambda i: (0, i, 0)),
            pl.BlockSpec((NC, _RBF, 16), lambda i: (0, i, 0)),
            pl.BlockSpec((264, NHID), lambda i: (0, 0)),
            pl.BlockSpec((1, NHID), lambda i: (0, 0)),
        ],
        out_specs=pl.BlockSpec((_RBF, NHID), lambda i: (i, 0)),
        out_shape=jax.ShapeDtypeStruct((N_NODES, NHID), jnp.float32),
    )(z2, sp2, deg2, w_comb, b2)


def kernel(x, edge_index, W1, b1, W2, b2):
    src = edge_index[0].astype(jnp.int32)
    dst = edge_index[1].astype(jnp.int32)

    ones16 = jnp.ones((_H_CH, 16), jnp.float32)
    zeros16 = jnp.zeros((_ZCH, 16), jnp.float32)
    zeros128 = jnp.zeros((_P_CH, _W), jnp.float32)

    srcx = jnp.concatenate([src, src + N_PAD])
    dstp = dst.reshape(NS, _P_CHUNKS, _P_CH)
    srch = src.reshape(NC, NS, _H_CHUNKS, _H_CH)
    dsth = dst.reshape(NC, NS, _H_CHUNKS, _H_CH)

    deg2 = _deg_kernel(dsth, ones16, zeros16)

    x_pad = jnp.pad(x, ((0, N_PAD - N_NODES), (0, 0)))
    table1, norm16, norm2_16 = _pre_call(x_pad, deg2)
    z1 = _prop_kernel(table1.reshape(NC * N_PAD, _W), srcx, dstp, zeros128,
                      norm2_16)
    sp2 = _sp_kernel(srch, dsth, norm16, zeros16)

    z2 = _prop_kernel(z1.reshape(NC * N_PAD, _W), srcx, dstp, zeros128,
                      norm16)

    a_pad = jnp.concatenate(
        [W1, b1[None, :], jnp.zeros((7, NHID), jnp.float32)], axis=0)
    w_comb = _wcomb_call(a_pad, W2)

    return _fin_call(z2, sp2, deg2, w_comb, b2[None, :])


# speedup vs baseline: 1.0825x; 1.0566x over previous
import functools

import jax
import jax.numpy as jnp
from jax import lax
from jax.experimental import pallas as pl
from jax.experimental.pallas import tpu as pltpu
from jax.experimental.pallas import tpu_sc as plsc

N_NODES = 10000
N_PAD = 10240
N_EDGES = 160000
NFEAT = 256
NHID = 512

NC = 2
NS = 16

_MESH = dict(core_axis_name="c", subcore_axis_name="s")
_SC_LINEAR = pltpu.CompilerParams(use_tc_tiling_on_sc=False)

_ROWS_PT = N_PAD // NS
_ZCH = 128


def _zero_acc(zeros_hbm, zbuf, acc, sid):
    pltpu.sync_copy(zeros_hbm, zbuf)
    for z in range(_ROWS_PT // _ZCH):
        pltpu.sync_copy(zbuf, acc.at[pl.ds(sid * _ROWS_PT + z * _ZCH, _ZCH)])


def _read_acc(out_hbm, zbuf, acc, cid, sid):
    for z in range(_ROWS_PT // _ZCH):
        r0 = sid * _ROWS_PT + z * _ZCH
        pltpu.sync_copy(acc.at[pl.ds(r0, _ZCH)], zbuf)
        pltpu.sync_copy(zbuf, out_hbm.at[cid].at[pl.ds(r0, _ZCH)])


def _edge_loop(n_chunks, gather_tab, src_all, dst_all, rows, gsem, ssem, acc):
    def start_gather(i, b):
        pltpu.async_copy(gather_tab.at[src_all.at[i]], rows[b], gsem[b])

    def wait_gather(i, b):
        pltpu.make_async_copy(gather_tab.at[src_all.at[i]], rows[b],
                              gsem[b]).wait()

    def start_scat(i, b):
        pltpu.async_copy(rows[b], acc.at[dst_all.at[i]], ssem[b], add=True)

    def wait_scat(i, b):
        pltpu.make_async_copy(rows[b], acc.at[dst_all.at[i]], ssem[b]).wait()

    start_gather(0, 0)
    start_gather(1, 1)
    wait_gather(0, 0)
    start_scat(0, 0)

    def body(io, carry):
        for k in range(2):
            i = io * 2 + 1 + k
            b = (1 + k) % 2
            wait_scat(i - 1, 1 - b)
            start_gather(i + 1, 1 - b)
            wait_gather(i, b)
            start_scat(i, b)
        return carry

    lax.fori_loop(0, (n_chunks - 3) // 2, body, 0)
    i = n_chunks - 2
    b = i % 2
    wait_scat(i - 1, 1 - b)
    start_gather(i + 1, 1 - b)
    wait_gather(i, b)
    start_scat(i, b)
    i = n_chunks - 1
    b = i % 2
    wait_gather(i, b)
    start_scat(i, b)
    wait_scat(i - 1, 1 - b)
    wait_scat(i, b)


_H_CH = 40
_H_EDGES_PT = N_EDGES // (NC * NS)
_H_CHUNKS = _H_EDGES_PT // _H_CH


def _deg_body(dst_hbm, ones_hbm, zeros_hbm, out_hbm,
              dst_all, ones_v, zbuf, acc, sem0, sem1):
    cid = lax.axis_index("c")
    sid = lax.axis_index("s")
    _zero_acc(zeros_hbm, zbuf, acc, sid)
    pltpu.sync_copy(ones_hbm, ones_v)
    pltpu.sync_copy(dst_hbm.at[cid].at[sid], dst_all)
    plsc.subcore_barrier()
    sems = (sem0, sem1)

    def start_sc(i, b):
        pltpu.async_copy(ones_v, acc.at[dst_all.at[i]], sems[b], add=True)

    def wait_sc(i, b):
        pltpu.make_async_copy(ones_v, acc.at[dst_all.at[i]], sems[b]).wait()

    start_sc(0, 0)
    start_sc(1, 1)

    def body(io, carry):
        for k in range(2):
            i = io * 2 + 2 + k
            b = k % 2
            wait_sc(i - 2, b)
            start_sc(i, b)
        return carry

    lax.fori_loop(0, (_H_CHUNKS - 3) // 2, body, 0)
    i = _H_CHUNKS - 1
    wait_sc(i - 2, i % 2)
    start_sc(i, i % 2)
    wait_sc(i - 1, (i - 1) % 2)
    wait_sc(i, i % 2)
    plsc.subcore_barrier()
    _read_acc(out_hbm, zbuf, acc, cid, sid)


_deg_kernel = functools.partial(
    pl.kernel,
    out_type=jax.ShapeDtypeStruct((NC, N_PAD, 16), jnp.float32),
    mesh=plsc.VectorSubcoreMesh(**_MESH),
    compiler_params=_SC_LINEAR,
    scratch_types=[
        pltpu.VMEM((_H_CHUNKS, _H_CH), jnp.int32),
        pltpu.VMEM((_H_CH, 16), jnp.float32),
        pltpu.VMEM((_ZCH, 16), jnp.float32),
        pltpu.VMEM_SHARED((N_PAD, 16), jnp.float32),
        pltpu.SemaphoreType.DMA,
        pltpu.SemaphoreType.DMA,
    ],
)(_deg_body)


def _sp_body(src_hbm, dst_hbm, norm16_hbm, zeros_hbm, out_hbm,
             src_all, dst_all, rows0, rows1, zbuf, acc, g0, g1, sc0, sc1):
    cid = lax.axis_index("c")
    sid = lax.axis_index("s")
    _zero_acc(zeros_hbm, zbuf, acc, sid)
    pltpu.sync_copy(src_hbm.at[cid].at[sid], src_all)
    pltpu.sync_copy(dst_hbm.at[cid].at[sid], dst_all)
    plsc.subcore_barrier()
    _edge_loop(_H_CHUNKS, norm16_hbm, src_all, dst_all,
               (rows0, rows1), (g0, g1), (sc0, sc1), acc)
    plsc.subcore_barrier()
    _read_acc(out_hbm, zbuf, acc, cid, sid)


_sp_kernel = functools.partial(
    pl.kernel,
    out_type=jax.ShapeDtypeStruct((NC, N_PAD, 16), jnp.float32),
    mesh=plsc.VectorSubcoreMesh(**_MESH),
    compiler_params=_SC_LINEAR,
    scratch_types=[
        pltpu.VMEM((_H_CHUNKS, _H_CH), jnp.int32),
        pltpu.VMEM((_H_CHUNKS, _H_CH), jnp.int32),
        pltpu.VMEM((_H_CH, 16), jnp.float32),
        pltpu.VMEM((_H_CH, 16), jnp.float32),
        pltpu.VMEM((_ZCH, 16), jnp.float32),
        pltpu.VMEM_SHARED((N_PAD, 16), jnp.float32),
        pltpu.SemaphoreType.DMA,
        pltpu.SemaphoreType.DMA,
        pltpu.SemaphoreType.DMA,
        pltpu.SemaphoreType.DMA,
    ],
)(_sp_body)


_P_CH = 80
_P_EDGES_PT = N_EDGES // NS
_P_CHUNKS = _P_EDGES_PT // _P_CH
_W = 128


def _prop_body(table_hbm, srcx_hbm, dst_hbm, zeros_hbm, out_hbm,
               dst_all, src0, src1, rows0, rows1, acc,
               s0, s1, g0, g1, c0, c1):
    cid = lax.axis_index("c")
    sid = lax.axis_index("s")
    srcs = (src0, src1)
    rows = (rows0, rows1)
    ssem = (s0, s1)
    gsem = (g0, g1)
    csem = (c0, c1)
    NZ = _ROWS_PT // _P_CH

    def zslice(z):
        return acc.at[pl.ds(sid * _ROWS_PT + z * _P_CH, _P_CH)]

    pltpu.sync_copy(zeros_hbm, rows0)
    for z in range(NZ):
        pltpu.async_copy(rows0, zslice(z), g0)
    pltpu.sync_copy(dst_hbm.at[sid], dst_all)
    for z in range(NZ):
        pltpu.make_async_copy(rows0, zslice(z), g0).wait()
    plsc.subcore_barrier()

    def src_slice(i):
        base = cid * N_EDGES + sid * _P_EDGES_PT + i * _P_CH
        return srcx_hbm.at[pl.ds(base, _P_CH)]

    def start_src(i, b):
        pltpu.async_copy(src_slice(i), srcs[b], ssem[b])

    def wait_src(i, b):
        pltpu.make_async_copy(src_slice(i), srcs[b], ssem[b]).wait()

    def start_gather(b):
        pltpu.async_copy(table_hbm.at[srcs[b]], rows[b], gsem[b])

    def wait_gather(b):
        pltpu.make_async_copy(table_hbm.at[srcs[b]], rows[b], gsem[b]).wait()

    def start_scat(i, b):
        pltpu.async_copy(rows[b], acc.at[dst_all.at[i]], csem[b], add=True)

    def wait_scat(i, b):
        pltpu.make_async_copy(rows[b], acc.at[dst_all.at[i]], csem[b]).wait()

    start_src(0, 0)
    wait_src(0, 0)
    start_gather(0)
    start_src(1, 1)
    wait_src(1, 1)
    start_gather(1)
    wait_gather(0)
    start_src(2, 0)
    start_scat(0, 0)

    def body(io, carry):
        for k in range(2):
            i = io * 2 + 1 + k
            b = (1 + k) % 2
            wait_src(i + 1, 1 - b)
            wait_scat(i - 1, 1 - b)
            start_gather(1 - b)
            wait_gather(b)
            start_src(i + 2, b)
            start_scat(i, b)
        return carry

    lax.fori_loop(0, (_P_CHUNKS - 3) // 2, body, 0)
    i = _P_CHUNKS - 2
    b = i % 2
    wait_src(i + 1, 1 - b)
    wait_scat(i - 1, 1 - b)
    start_gather(1 - b)
    wait_gather(b)
    start_scat(i, b)
    i = _P_CHUNKS - 1
    b = i % 2
    wait_gather(b)
    start_scat(i, b)
    wait_scat(i - 1, 1 - b)
    wait_scat(i, b)
    plsc.subcore_barrier()

    def oslice(z):
        return out_hbm.at[cid].at[pl.ds(sid * _ROWS_PT + z * _P_CH, _P_CH)]

    for z in range(NZ):
        b = z % 2
        if z >= 2:
            pltpu.make_async_copy(rows[b], oslice(z - 2), gsem[b]).wait()
        pltpu.sync_copy(zslice(z), rows[b])
        pltpu.async_copy(rows[b], oslice(z), gsem[b])
    pltpu.make_async_copy(rows[0], oslice(NZ - 2), gsem[0]).wait()
    pltpu.make_async_copy(rows[1], oslice(NZ - 1), gsem[1]).wait()


_prop_kernel = functools.partial(
    pl.kernel,
    out_type=jax.ShapeDtypeStruct((NC, N_PAD, _W), jnp.float32),
    mesh=plsc.VectorSubcoreMesh(**_MESH),
    scratch_types=[
        pltpu.VMEM((_P_CHUNKS, _P_CH), jnp.int32),
        pltpu.VMEM((_P_CH,), jnp.int32),
        pltpu.VMEM((_P_CH,), jnp.int32),
        pltpu.VMEM((_P_CH, _W), jnp.float32),
        pltpu.VMEM((_P_CH, _W), jnp.float32),
        pltpu.VMEM_SHARED((N_PAD, _W), jnp.float32),
        pltpu.SemaphoreType.DMA,
        pltpu.SemaphoreType.DMA,
        pltpu.SemaphoreType.DMA,
        pltpu.SemaphoreType.DMA,
        pltpu.SemaphoreType.DMA,
        pltpu.SemaphoreType.DMA,
    ],
)(_prop_body)


_RB = 2048
_RBF = 2000


def _norm_from_deg(deg_ref):
    deg = deg_ref[0, :, 0:1] + deg_ref[1, :, 0:1]
    return lax.rsqrt(jnp.maximum(deg, 1.0))


def _pre_body(x_ref, deg_ref, t_ref, n_ref):
    norm = _norm_from_deg(deg_ref)
    y = x_ref[...] * norm
    t_ref[0] = y[:, 0:128]
    t_ref[1] = y[:, 128:256]
    n_ref[...] = jnp.broadcast_to(norm, (norm.shape[0], 16))


def _pre_call(x, deg2):
    return pl.pallas_call(
        _pre_body,
        grid=(N_PAD // _RB,),
        in_specs=[
            pl.BlockSpec((_RB, NFEAT), lambda i: (i, 0)),
            pl.BlockSpec((NC, _RB, 16), lambda i: (0, i, 0)),
        ],
        out_specs=[
            pl.BlockSpec((NC, _RB, 128), lambda i: (0, i, 0)),
            pl.BlockSpec((_RB, 16), lambda i: (i, 0)),
        ],
        out_shape=[
            jax.ShapeDtypeStruct((NC, N_PAD, 128), jnp.float32),
            jax.ShapeDtypeStruct((N_PAD, 16), jnp.float32),
        ],
    )(x, deg2)


def _mid_body(z_ref, deg_ref, t_ref):
    norm = _norm_from_deg(deg_ref)
    inv = norm * norm
    t_ref[0] = z_ref[0] * inv
    t_ref[1] = z_ref[1] * inv


def _mid_call(z1, deg2):
    return pl.pallas_call(
        _mid_body,
        grid=(N_PAD // _RB,),
        in_specs=[
            pl.BlockSpec((NC, _RB, 128), lambda i: (0, i, 0)),
            pl.BlockSpec((NC, _RB, 16), lambda i: (0, i, 0)),
        ],
        out_specs=pl.BlockSpec((NC, _RB, 128), lambda i: (0, i, 0)),
        out_shape=jax.ShapeDtypeStruct((NC, N_PAD, 128), jnp.float32),
    )(z1, deg2)


def _wcomb_body(a_ref, w2_ref, o_ref):
    o_ref[...] = jnp.dot(a_ref[...], w2_ref[...],
                         preferred_element_type=jnp.float32)


def _wcomb_call(a_pad, w2):
    return pl.pallas_call(
        _wcomb_body,
        out_shape=jax.ShapeDtypeStruct((264, NHID), jnp.float32),
    )(a_pad, w2)


def _fin_body(z2_ref, sp_ref, deg_ref, w_ref, b2_ref, o_ref):
    norm = _norm_from_deg(deg_ref)
    h0 = z2_ref[0] * norm
    h1 = z2_ref[1] * norm
    u = norm * (sp_ref[0, :, 0:1] + sp_ref[1, :, 0:1])
    acc = jnp.dot(h0, w_ref[0:128], preferred_element_type=jnp.float32)
    acc = acc + jnp.dot(h1, w_ref[128:256], preferred_element_type=jnp.float32)
    acc = acc + u * w_ref[256:257]
    o_ref[...] = acc + b2_ref[...]


def _fin_call(z2, sp2, deg2, w_comb, b2):
    return pl.pallas_call(
        _fin_body,
        grid=(N_NODES // _RBF,),
        in_specs=[
            pl.BlockSpec((NC, _RBF, 128), lambda i: (0, i, 0)),
            pl.BlockSpec((NC, _RBF, 16), lambda i: (0, i, 0)),
            pl.BlockSpec((NC, _RBF, 16), lambda i: (0, i, 0)),
            pl.BlockSpec((264, NHID), lambda i: (0, 0)),
            pl.BlockSpec((1, NHID), lambda i: (0, 0)),
        ],
        out_specs=pl.BlockSpec((_RBF, NHID), lambda i: (i, 0)),
        out_shape=jax.ShapeDtypeStruct((N_NODES, NHID), jnp.float32),
    )(z2, sp2, deg2, w_comb, b2)


def kernel(x, edge_index, W1, b1, W2, b2):
    src = edge_index[0].astype(jnp.int32)
    dst = edge_index[1].astype(jnp.int32)

    ones16 = jnp.ones((_H_CH, 16), jnp.float32)
    zeros16 = jnp.zeros((_ZCH, 16), jnp.float32)
    zeros128 = jnp.zeros((_P_CH, _W), jnp.float32)

    srcx = jnp.concatenate([src, src + N_PAD])
    dstp = dst.reshape(NS, _P_CHUNKS, _P_CH)
    srch = src.reshape(NC, NS, _H_CHUNKS, _H_CH)
    dsth = dst.reshape(NC, NS, _H_CHUNKS, _H_CH)

    deg2 = _deg_kernel(dsth, ones16, zeros16)

    x_pad = jnp.pad(x, ((0, N_PAD - N_NODES), (0, 0)))
    table1, norm16 = _pre_call(x_pad, deg2)
    z1 = _prop_kernel(table1.reshape(NC * N_PAD, _W), srcx, dstp, zeros128)
    sp2 = _sp_kernel(srch, dsth, norm16, zeros16)

    table2 = _mid_call(z1, deg2)
    z2 = _prop_kernel(table2.reshape(NC * N_PAD, _W), srcx, dstp, zeros128)

    a_pad = jnp.concatenate(
        [W1, b1[None, :], jnp.zeros((7, NHID), jnp.float32)], axis=0)
    w_comb = _wcomb_call(a_pad, W2)

    return _fin_call(z2, sp2, deg2, w_comb, b2[None, :])


# speedup vs baseline: 1.1582x; 1.0698x over previous
import functools

import jax
import jax.numpy as jnp
from jax import lax
from jax.experimental import pallas as pl
from jax.experimental.pallas import tpu as pltpu
from jax.experimental.pallas import tpu_sc as plsc

N_NODES = 10000
N_PAD = 10240
N_EDGES = 160000
NFEAT = 256
NHID = 512

NC = 2
NS = 16

_MESH = dict(core_axis_name="c", subcore_axis_name="s")
_SC_LINEAR = pltpu.CompilerParams(use_tc_tiling_on_sc=False)

_ROWS_PT = N_PAD // NS
_ZCH = 128


def _zero_acc(zeros_hbm, zbuf, acc, sid):
    pltpu.sync_copy(zeros_hbm, zbuf)
    for z in range(_ROWS_PT // _ZCH):
        pltpu.sync_copy(zbuf, acc.at[pl.ds(sid * _ROWS_PT + z * _ZCH, _ZCH)])


def _read_acc(out_hbm, zbuf, acc, cid, sid):
    for z in range(_ROWS_PT // _ZCH):
        r0 = sid * _ROWS_PT + z * _ZCH
        pltpu.sync_copy(acc.at[pl.ds(r0, _ZCH)], zbuf)
        pltpu.sync_copy(zbuf, out_hbm.at[cid].at[pl.ds(r0, _ZCH)])


def _edge_loop(n_chunks, gather_tab, src_all, dst_all, rows, gsem, ssem, acc):
    def start_gather(i, b):
        pltpu.async_copy(gather_tab.at[src_all.at[i]], rows[b], gsem[b])

    def wait_gather(i, b):
        pltpu.make_async_copy(gather_tab.at[src_all.at[i]], rows[b],
                              gsem[b]).wait()

    def start_scat(i, b):
        pltpu.async_copy(rows[b], acc.at[dst_all.at[i]], ssem[b], add=True)

    def wait_scat(i, b):
        pltpu.make_async_copy(rows[b], acc.at[dst_all.at[i]], ssem[b]).wait()

    start_gather(0, 0)
    start_gather(1, 1)
    wait_gather(0, 0)
    start_scat(0, 0)

    def body(io, carry):
        for k in range(2):
            i = io * 2 + 1 + k
            b = (1 + k) % 2
            wait_scat(i - 1, 1 - b)
            start_gather(i + 1, 1 - b)
            wait_gather(i, b)
            start_scat(i, b)
        return carry

    lax.fori_loop(0, (n_chunks - 3) // 2, body, 0)
    i = n_chunks - 2
    b = i % 2
    wait_scat(i - 1, 1 - b)
    start_gather(i + 1, 1 - b)
    wait_gather(i, b)
    start_scat(i, b)
    i = n_chunks - 1
    b = i % 2
    wait_gather(i, b)
    start_scat(i, b)
    wait_scat(i - 1, 1 - b)
    wait_scat(i, b)


_H_CH = 40
_H_EDGES_PT = N_EDGES // (NC * NS)
_H_CHUNKS = _H_EDGES_PT // _H_CH


def _deg_body(dst_hbm, ones_hbm, zeros_hbm, out_hbm,
              dst_all, ones_v, zbuf, acc, sem0, sem1):
    cid = lax.axis_index("c")
    sid = lax.axis_index("s")
    _zero_acc(zeros_hbm, zbuf, acc, sid)
    pltpu.sync_copy(ones_hbm, ones_v)
    pltpu.sync_copy(dst_hbm.at[cid].at[sid], dst_all)
    plsc.subcore_barrier()
    sems = (sem0, sem1)

    def start_sc(i, b):
        pltpu.async_copy(ones_v, acc.at[dst_all.at[i]], sems[b], add=True)

    def wait_sc(i, b):
        pltpu.make_async_copy(ones_v, acc.at[dst_all.at[i]], sems[b]).wait()

    start_sc(0, 0)
    start_sc(1, 1)

    def body(io, carry):
        for k in range(2):
            i = io * 2 + 2 + k
            b = k % 2
            wait_sc(i - 2, b)
            start_sc(i, b)
        return carry

    lax.fori_loop(0, (_H_CHUNKS - 3) // 2, body, 0)
    i = _H_CHUNKS - 1
    wait_sc(i - 2, i % 2)
    start_sc(i, i % 2)
    wait_sc(i - 1, (i - 1) % 2)
    wait_sc(i, i % 2)
    plsc.subcore_barrier()
    _read_acc(out_hbm, zbuf, acc, cid, sid)


_deg_kernel = functools.partial(
    pl.kernel,
    out_type=jax.ShapeDtypeStruct((NC, N_PAD, 16), jnp.float32),
    mesh=plsc.VectorSubcoreMesh(**_MESH),
    compiler_params=_SC_LINEAR,
    scratch_types=[
        pltpu.VMEM((_H_CHUNKS, _H_CH), jnp.int32),
        pltpu.VMEM((_H_CH, 16), jnp.float32),
        pltpu.VMEM((_ZCH, 16), jnp.float32),
        pltpu.VMEM_SHARED((N_PAD, 16), jnp.float32),
        pltpu.SemaphoreType.DMA,
        pltpu.SemaphoreType.DMA,
    ],
)(_deg_body)


def _sp_body(src_hbm, dst_hbm, norm16_hbm, zeros_hbm, out_hbm,
             src_all, dst_all, rows0, rows1, zbuf, acc, normsh,
             g0, g1, sc0, sc1):
    cid = lax.axis_index("c")
    sid = lax.axis_index("s")
    _zero_acc(zeros_hbm, zbuf, acc, sid)
    for z in range(_ROWS_PT // _ZCH):
        r0 = sid * _ROWS_PT + z * _ZCH
        pltpu.sync_copy(norm16_hbm.at[pl.ds(r0, _ZCH)], zbuf)
        pltpu.sync_copy(zbuf, normsh.at[pl.ds(r0, _ZCH)])
    pltpu.sync_copy(src_hbm.at[cid].at[sid], src_all)
    pltpu.sync_copy(dst_hbm.at[cid].at[sid], dst_all)
    plsc.subcore_barrier()
    _edge_loop(_H_CHUNKS, normsh, src_all, dst_all,
               (rows0, rows1), (g0, g1), (sc0, sc1), acc)
    plsc.subcore_barrier()
    _read_acc(out_hbm, zbuf, acc, cid, sid)


_sp_kernel = functools.partial(
    pl.kernel,
    out_type=jax.ShapeDtypeStruct((NC, N_PAD, 16), jnp.float32),
    mesh=plsc.VectorSubcoreMesh(**_MESH),
    compiler_params=_SC_LINEAR,
    scratch_types=[
        pltpu.VMEM((_H_CHUNKS, _H_CH), jnp.int32),
        pltpu.VMEM((_H_CHUNKS, _H_CH), jnp.int32),
        pltpu.VMEM((_H_CH, 16), jnp.float32),
        pltpu.VMEM((_H_CH, 16), jnp.float32),
        pltpu.VMEM((_ZCH, 16), jnp.float32),
        pltpu.VMEM_SHARED((N_PAD, 16), jnp.float32),
        pltpu.VMEM_SHARED((N_PAD, 16), jnp.float32),
        pltpu.SemaphoreType.DMA,
        pltpu.SemaphoreType.DMA,
        pltpu.SemaphoreType.DMA,
        pltpu.SemaphoreType.DMA,
    ],
)(_sp_body)


_P_CH = 80
_P_EDGES_PT = N_EDGES // NS
_P_CHUNKS = _P_EDGES_PT // _P_CH
_W = 128


def _prop_body(table_hbm, srcx_hbm, dst_hbm, zeros_hbm, out_hbm,
               dst_all, src0, src1, rows0, rows1, acc,
               s0, s1, g0, g1, c0, c1):
    cid = lax.axis_index("c")
    sid = lax.axis_index("s")
    srcs = (src0, src1)
    rows = (rows0, rows1)
    ssem = (s0, s1)
    gsem = (g0, g1)
    csem = (c0, c1)
    NZ = _ROWS_PT // _P_CH

    def zslice(z):
        return acc.at[pl.ds(sid * _ROWS_PT + z * _P_CH, _P_CH)]

    pltpu.sync_copy(zeros_hbm, rows0)
    for z in range(NZ):
        pltpu.async_copy(rows0, zslice(z), g0)
    pltpu.sync_copy(dst_hbm.at[sid], dst_all)
    for z in range(NZ):
        pltpu.make_async_copy(rows0, zslice(z), g0).wait()
    plsc.subcore_barrier()

    def src_slice(i):
        base = cid * N_EDGES + sid * _P_EDGES_PT + i * _P_CH
        return srcx_hbm.at[pl.ds(base, _P_CH)]

    def start_src(i, b):
        pltpu.async_copy(src_slice(i), srcs[b], ssem[b])

    def wait_src(i, b):
        pltpu.make_async_copy(src_slice(i), srcs[b], ssem[b]).wait()

    def start_gather(b):
        pltpu.async_copy(table_hbm.at[srcs[b]], rows[b], gsem[b])

    def wait_gather(b):
        pltpu.make_async_copy(table_hbm.at[srcs[b]], rows[b], gsem[b]).wait()

    def start_scat(i, b):
        pltpu.async_copy(rows[b], acc.at[dst_all.at[i]], csem[b], add=True)

    def wait_scat(i, b):
        pltpu.make_async_copy(rows[b], acc.at[dst_all.at[i]], csem[b]).wait()

    start_src(0, 0)
    wait_src(0, 0)
    start_gather(0)
    start_src(1, 1)
    wait_src(1, 1)
    start_gather(1)
    wait_gather(0)
    start_src(2, 0)
    start_scat(0, 0)

    def body(io, carry):
        for k in range(2):
            i = io * 2 + 1 + k
            b = (1 + k) % 2
            wait_src(i + 1, 1 - b)
            wait_scat(i - 1, 1 - b)
            start_gather(1 - b)
            wait_gather(b)
            start_src(i + 2, b)
            start_scat(i, b)
        return carry

    lax.fori_loop(0, (_P_CHUNKS - 3) // 2, body, 0)
    i = _P_CHUNKS - 2
    b = i % 2
    wait_src(i + 1, 1 - b)
    wait_scat(i - 1, 1 - b)
    start_gather(1 - b)
    wait_gather(b)
    start_scat(i, b)
    i = _P_CHUNKS - 1
    b = i % 2
    wait_gather(b)
    start_scat(i, b)
    wait_scat(i - 1, 1 - b)
    wait_scat(i, b)
    plsc.subcore_barrier()

    def oslice(z):
        return out_hbm.at[cid].at[pl.ds(sid * _ROWS_PT + z * _P_CH, _P_CH)]

    for z in range(NZ):
        b = z % 2
        if z >= 2:
            pltpu.make_async_copy(rows[b], oslice(z - 2), gsem[b]).wait()
        pltpu.sync_copy(zslice(z), rows[b])
        pltpu.async_copy(rows[b], oslice(z), gsem[b])
    pltpu.make_async_copy(rows[0], oslice(NZ - 2), gsem[0]).wait()
    pltpu.make_async_copy(rows[1], oslice(NZ - 1), gsem[1]).wait()


_prop_kernel = functools.partial(
    pl.kernel,
    out_type=jax.ShapeDtypeStruct((NC, N_PAD, _W), jnp.float32),
    mesh=plsc.VectorSubcoreMesh(**_MESH),
    scratch_types=[
        pltpu.VMEM((_P_CHUNKS, _P_CH), jnp.int32),
        pltpu.VMEM((_P_CH,), jnp.int32),
        pltpu.VMEM((_P_CH,), jnp.int32),
        pltpu.VMEM((_P_CH, _W), jnp.float32),
        pltpu.VMEM((_P_CH, _W), jnp.float32),
        pltpu.VMEM_SHARED((N_PAD, _W), jnp.float32),
        pltpu.SemaphoreType.DMA,
        pltpu.SemaphoreType.DMA,
        pltpu.SemaphoreType.DMA,
        pltpu.SemaphoreType.DMA,
        pltpu.SemaphoreType.DMA,
        pltpu.SemaphoreType.DMA,
    ],
)(_prop_body)


_RB = 2048
_RBF = 2000


def _norm_from_deg(deg_ref):
    deg = deg_ref[0, :, 0:1] + deg_ref[1, :, 0:1]
    return lax.rsqrt(jnp.maximum(deg, 1.0))


def _pre_body(x_ref, deg_ref, t_ref, n_ref):
    norm = _norm_from_deg(deg_ref)
    y = x_ref[...] * norm
    t_ref[0] = y[:, 0:128]
    t_ref[1] = y[:, 128:256]
    n_ref[...] = jnp.broadcast_to(norm, (norm.shape[0], 16))


def _pre_call(x, deg2):
    return pl.pallas_call(
        _pre_body,
        grid=(N_PAD // _RB,),
        in_specs=[
            pl.BlockSpec((_RB, NFEAT), lambda i: (i, 0)),
            pl.BlockSpec((NC, _RB, 16), lambda i: (0, i, 0)),
        ],
        out_specs=[
            pl.BlockSpec((NC, _RB, 128), lambda i: (0, i, 0)),
            pl.BlockSpec((_RB, 16), lambda i: (i, 0)),
        ],
        out_shape=[
            jax.ShapeDtypeStruct((NC, N_PAD, 128), jnp.float32),
            jax.ShapeDtypeStruct((N_PAD, 16), jnp.float32),
        ],
    )(x, deg2)


def _mid_body(z_ref, deg_ref, t_ref):
    norm = _norm_from_deg(deg_ref)
    inv = norm * norm
    t_ref[0] = z_ref[0] * inv
    t_ref[1] = z_ref[1] * inv


def _mid_call(z1, deg2):
    return pl.pallas_call(
        _mid_body,
        grid=(N_PAD // _RB,),
        in_specs=[
            pl.BlockSpec((NC, _RB, 128), lambda i: (0, i, 0)),
            pl.BlockSpec((NC, _RB, 16), lambda i: (0, i, 0)),
        ],
        out_specs=pl.BlockSpec((NC, _RB, 128), lambda i: (0, i, 0)),
        out_shape=jax.ShapeDtypeStruct((NC, N_PAD, 128), jnp.float32),
    )(z1, deg2)


def _fin_body(z2_ref, sp_ref, deg_ref, a_ref, w2_ref, b2_ref, o_ref, w_ref):
    @pl.when(pl.program_id(0) == 0)
    def _():
        w_ref[...] = jnp.dot(a_ref[...], w2_ref[...],
                             preferred_element_type=jnp.float32)

    norm = _norm_from_deg(deg_ref)
    h0 = z2_ref[0] * norm
    h1 = z2_ref[1] * norm
    u = norm * (sp_ref[0, :, 0:1] + sp_ref[1, :, 0:1])
    acc = jnp.dot(h0, w_ref[0:128], preferred_element_type=jnp.float32)
    acc = acc + jnp.dot(h1, w_ref[128:256], preferred_element_type=jnp.float32)
    acc = acc + u * w_ref[256:257]
    o_ref[...] = acc + b2_ref[...]


def _fin_call(z2, sp2, deg2, a_pad, w2, b2):
    return pl.pallas_call(
        _fin_body,
        grid=(N_NODES // _RBF,),
        in_specs=[
            pl.BlockSpec((NC, _RBF, 128), lambda i: (0, i, 0)),
            pl.BlockSpec((NC, _RBF, 16), lambda i: (0, i, 0)),
            pl.BlockSpec((NC, _RBF, 16), lambda i: (0, i, 0)),
            pl.BlockSpec((264, NHID), lambda i: (0, 0)),
            pl.BlockSpec((NHID, NHID), lambda i: (0, 0)),
            pl.BlockSpec((1, NHID), lambda i: (0, 0)),
        ],
        out_specs=pl.BlockSpec((_RBF, NHID), lambda i: (i, 0)),
        out_shape=jax.ShapeDtypeStruct((N_NODES, NHID), jnp.float32),
        scratch_shapes=[pltpu.VMEM((264, NHID), jnp.float32)],
    )(z2, sp2, deg2, a_pad, w2, b2)


def kernel(x, edge_index, W1, b1, W2, b2):
    src = edge_index[0].astype(jnp.int32)
    dst = edge_index[1].astype(jnp.int32)

    ones16 = jnp.ones((_H_CH, 16), jnp.float32)
    zeros16 = jnp.zeros((_ZCH, 16), jnp.float32)
    zeros128 = jnp.zeros((_P_CH, _W), jnp.float32)

    srcx = jnp.concatenate([src, src + N_PAD])
    dstp = dst.reshape(NS, _P_CHUNKS, _P_CH)
    srch = src.reshape(NC, NS, _H_CHUNKS, _H_CH)
    dsth = dst.reshape(NC, NS, _H_CHUNKS, _H_CH)

    deg2 = _deg_kernel(dsth, ones16, zeros16)

    x_pad = jnp.pad(x, ((0, N_PAD - N_NODES), (0, 0)))
    table1, norm16 = _pre_call(x_pad, deg2)
    z1 = _prop_kernel(table1.reshape(NC * N_PAD, _W), srcx, dstp, zeros128)
    sp2 = _sp_kernel(srch, dsth, norm16, zeros16)

    table2 = _mid_call(z1, deg2)
    z2 = _prop_kernel(table2.reshape(NC * N_PAD, _W), srcx, dstp, zeros128)

    a_pad = jnp.concatenate(
        [W1, b1[None, :], jnp.zeros((7, NHID), jnp.float32)], axis=0)

    return _fin_call(z2, sp2, deg2, a_pad, W2, b2[None, :])
